# Initial kernel scaffold; baseline (speedup 1.0000x reference)
import jax
import jax.numpy as jnp
from jax.experimental import pallas as pl


def kernel(x, edge_index, W1l, b1l, W1r, W2l, b2l, W2r):
    raise NotImplementedError("write your pallas kernel here")


# speedup vs baseline: 3.6104x; 3.6104x over previous
import functools

import jax
import jax.numpy as jnp
from jax import lax
from jax.experimental import pallas as pl
from jax.experimental.pallas import tpu as pltpu
from jax.experimental.pallas import tpu_sc as plsc

_N = 10000
_E = 320000
_D = 128
_NC = 2
_NS = 16
_NW = _NC * _NS
_EPT = _E // _NW
_C = 80
_NCHUNK = _EPT // _C
_RPT = 624
_RPT_LAST = _N - 15 * _RPT
_BS = 1000
_G = _N // _BS


def _make_sc_agg(with_deg: bool):
    mesh = plsc.VectorSubcoreMesh(core_axis_name="c", subcore_axis_name="s")
    acc_type = jax.ShapeDtypeStruct((_NC * _N, _D), jnp.float32)
    out_type = [acc_type] if with_deg else acc_type
    scratch = [
        pltpu.VMEM((_C,), jnp.int32),
        pltpu.VMEM((_C,), jnp.int32),
        pltpu.VMEM((_C, _D), jnp.float32),
        pltpu.VMEM_SHARED((_N, _D), jnp.float32),
        pltpu.SemaphoreType.DMA,
    ]
    if with_deg:
        out_type.append(jax.ShapeDtypeStruct((_NC * _N, 16), jnp.float32))
        scratch += [
            pltpu.VMEM((_C, 16), jnp.float32),
            pltpu.VMEM_SHARED((_N, 16), jnp.float32),
        ]

    def body(*refs):
        if with_deg:
            (y_hbm, src_hbm, dst_hbm, z128_hbm, z16_hbm, ones_hbm,
             acc_out, deg_out,
             src_v, dst_v, rows_v, acc_sh, sem, ones_v, deg_sh) = refs
        else:
            (y_hbm, src_hbm, dst_hbm, z128_hbm,
             acc_out,
             src_v, dst_v, rows_v, acc_sh, sem) = refs
        c = lax.axis_index("c")
        s = lax.axis_index("s")
        gid = c * _NS + s
        row0 = s * _RPT

        @pl.when(s < _NS - 1)
        def _():
            pltpu.sync_copy(z128_hbm.at[pl.ds(0, _RPT)],
                            acc_sh.at[pl.ds(row0, _RPT)])
            if with_deg:
                pltpu.sync_copy(z16_hbm.at[pl.ds(0, _RPT)],
                                deg_sh.at[pl.ds(row0, _RPT)])

        @pl.when(s == _NS - 1)
        def _():
            pltpu.sync_copy(z128_hbm, acc_sh.at[pl.ds(15 * _RPT, _RPT_LAST)])
            if with_deg:
                pltpu.sync_copy(z16_hbm,
                                deg_sh.at[pl.ds(15 * _RPT, _RPT_LAST)])

        if with_deg:
            pltpu.sync_copy(ones_hbm, ones_v)
        plsc.subcore_barrier()

        base = gid * _EPT

        def chunk(k, carry):
            off = base + k * _C
            pltpu.sync_copy(src_hbm.at[pl.ds(off, _C)], src_v)
            pltpu.sync_copy(dst_hbm.at[pl.ds(off, _C)], dst_v)
            pltpu.async_copy(y_hbm.at[src_v], rows_v, sem).wait()
            pltpu.sync_copy(rows_v, acc_sh.at[dst_v], add=True)
            if with_deg:
                pltpu.sync_copy(ones_v, deg_sh.at[dst_v], add=True)
            return carry

        lax.fori_loop(0, _NCHUNK, chunk, 0)
        plsc.subcore_barrier()
        out0 = c * _N + row0

        @pl.when(s < _NS - 1)
        def _():
            pltpu.sync_copy(acc_sh.at[pl.ds(row0, _RPT)],
                            acc_out.at[pl.ds(out0, _RPT)])
            if with_deg:
                pltpu.sync_copy(deg_sh.at[pl.ds(row0, _RPT)],
                                deg_out.at[pl.ds(out0, _RPT)])

        @pl.when(s == _NS - 1)
        def _():
            last0 = c * _N + 15 * _RPT
            pltpu.sync_copy(acc_sh.at[pl.ds(15 * _RPT, _RPT_LAST)],
                            acc_out.at[pl.ds(last0, _RPT_LAST)])
            if with_deg:
                pltpu.sync_copy(deg_sh.at[pl.ds(15 * _RPT, _RPT_LAST)],
                                deg_out.at[pl.ds(last0, _RPT_LAST)])

    return pl.kernel(body, out_type=out_type, mesh=mesh,
                     scratch_types=scratch)


_sc_agg_deg = _make_sc_agg(True)
_sc_agg = _make_sc_agg(False)


def _dot_t(a, w):
    return lax.dot_general(a, w, (((1,), (1,)), ((), ())),
                           preferred_element_type=jnp.float32)


def _tc_pre_body(x_ref, wl_ref, wr_ref, y_ref, r_ref):
    xb = x_ref[...]
    y_ref[...] = _dot_t(xb, wl_ref[...])
    r_ref[...] = _dot_t(xb, wr_ref[...])


@jax.jit
def _tc_pre(x, wl, wr):
    return pl.pallas_call(
        _tc_pre_body,
        grid=(_G,),
        in_specs=[
            pl.BlockSpec((_BS, _D), lambda i: (i, 0)),
            pl.BlockSpec((_D, _D), lambda i: (0, 0)),
            pl.BlockSpec((_D, _D), lambda i: (0, 0)),
        ],
        out_specs=[
            pl.BlockSpec((_BS, _D), lambda i: (i, 0)),
            pl.BlockSpec((_BS, _D), lambda i: (i, 0)),
        ],
        out_shape=[
            jax.ShapeDtypeStruct((_N, _D), jnp.float32),
            jax.ShapeDtypeStruct((_N, _D), jnp.float32),
        ],
    )(x, wl, wr)


def _tc_mid_body(aA_ref, aB_ref, dA_ref, dB_ref, r1_ref, b_ref,
                 wl_ref, wr_ref, y2_ref, r2_ref):
    ssum = aA_ref[...] + aB_ref[...]
    deg = dA_ref[:, 0:1] + dB_ref[:, 0:1]
    inv = 1.0 / jnp.maximum(deg, 1.0)
    h = jnp.maximum(ssum * inv + b_ref[...] + r1_ref[...], 0.0)
    y2_ref[...] = _dot_t(h, wl_ref[...])
    r2_ref[...] = _dot_t(h, wr_ref[...])


@jax.jit
def _tc_mid(acc, deg, r1, b, wl, wr):
    return pl.pallas_call(
        _tc_mid_body,
        grid=(_G,),
        in_specs=[
            pl.BlockSpec((_BS, _D), lambda i: (i, 0)),
            pl.BlockSpec((_BS, _D), lambda i: (i + _G, 0)),
            pl.BlockSpec((_BS, 16), lambda i: (i, 0)),
            pl.BlockSpec((_BS, 16), lambda i: (i + _G, 0)),
            pl.BlockSpec((_BS, _D), lambda i: (i, 0)),
            pl.BlockSpec((1, _D), lambda i: (0, 0)),
            pl.BlockSpec((_D, _D), lambda i: (0, 0)),
            pl.BlockSpec((_D, _D), lambda i: (0, 0)),
        ],
        out_specs=[
            pl.BlockSpec((_BS, _D), lambda i: (i, 0)),
            pl.BlockSpec((_BS, _D), lambda i: (i, 0)),
        ],
        out_shape=[
            jax.ShapeDtypeStruct((_N, _D), jnp.float32),
            jax.ShapeDtypeStruct((_N, _D), jnp.float32),
        ],
    )(acc, acc, deg, deg, r1, b, wl, wr)


def _tc_final_body(aA_ref, aB_ref, dA_ref, dB_ref, r2_ref, b_ref, z_ref):
    ssum = aA_ref[...] + aB_ref[...]
    deg = dA_ref[:, 0:1] + dB_ref[:, 0:1]
    inv = 1.0 / jnp.maximum(deg, 1.0)
    z_ref[...] = ssum * inv + b_ref[...] + r2_ref[...]


@jax.jit
def _tc_final(acc, deg, r2, b):
    return pl.pallas_call(
        _tc_final_body,
        grid=(_G,),
        in_specs=[
            pl.BlockSpec((_BS, _D), lambda i: (i, 0)),
            pl.BlockSpec((_BS, _D), lambda i: (i + _G, 0)),
            pl.BlockSpec((_BS, 16), lambda i: (i, 0)),
            pl.BlockSpec((_BS, 16), lambda i: (i + _G, 0)),
            pl.BlockSpec((_BS, _D), lambda i: (i, 0)),
            pl.BlockSpec((1, _D), lambda i: (0, 0)),
        ],
        out_specs=pl.BlockSpec((_BS, _D), lambda i: (i, 0)),
        out_shape=jax.ShapeDtypeStruct((_N, _D), jnp.float32),
    )(acc, acc, deg, deg, r2, b)


def kernel(x, edge_index, W1l, b1l, W1r, W2l, b2l, W2r):
    src = edge_index[0].astype(jnp.int32)
    dst = edge_index[1].astype(jnp.int32)
    z128 = jnp.zeros((_RPT_LAST, _D), jnp.float32)
    z16 = jnp.zeros((_RPT_LAST, 16), jnp.float32)
    ones = jnp.ones((_C, 16), jnp.float32)
    b1 = b1l.reshape(1, _D)
    b2 = b2l.reshape(1, _D)

    y1, r1 = _tc_pre(x, W1l, W1r)
    degv = jax.ops.segment_sum(jnp.ones((_E,), jnp.float32), dst,
                               num_segments=_N)
    deg = jnp.concatenate(
        [jnp.tile(degv[:, None], (1, 16)), jnp.zeros((_N, 16), jnp.float32)])
    acc1 = _sc_agg(y1, src, dst, z128)
    y2, r2 = _tc_mid(acc1, deg, r1, b1, W2l, W2r)
    acc2 = _sc_agg(y2, src, dst, z128)
    z = _tc_final(acc2, deg, r2, b2)
    return z


# speedup vs baseline: 4.7160x; 1.3063x over previous
import jax
import jax.numpy as jnp
from jax import lax
from jax.experimental import pallas as pl
from jax.experimental.pallas import tpu as pltpu
from jax.experimental.pallas import tpu_sc as plsc

_N = 10000
_E = 320000
_D = 128
_NC = 2
_NS = 16
_NW = _NC * _NS
_EPT = _E // _NW
_C = 80
_NCHUNK = _EPT // _C
_RPT = 624
_RPT_LAST = _N - 15 * _RPT
_BS = 1000
_G = _N // _BS

_MESH = plsc.VectorSubcoreMesh(core_axis_name="c", subcore_axis_name="s")


def _zero_init(s, z_hbm, sh):
    @pl.when(s < _NS - 1)
    def _():
        pltpu.sync_copy(z_hbm.at[pl.ds(0, _RPT)],
                        sh.at[pl.ds(s * _RPT, _RPT)])

    @pl.when(s == _NS - 1)
    def _():
        pltpu.sync_copy(z_hbm, sh.at[pl.ds(15 * _RPT, _RPT_LAST)])


def _write_out(c, s, sh, out):
    @pl.when(s < _NS - 1)
    def _():
        row0 = s * _RPT
        pltpu.sync_copy(sh.at[pl.ds(row0, _RPT)],
                        out.at[pl.ds(c * _N + row0, _RPT)])

    @pl.when(s == _NS - 1)
    def _():
        pltpu.sync_copy(sh.at[pl.ds(15 * _RPT, _RPT_LAST)],
                        out.at[pl.ds(c * _N + 15 * _RPT, _RPT_LAST)])


def _sc_agg_body(y_hbm, src_hbm, dst_hbm, z_hbm, acc_out,
                 src_v, dst_v, rows_v, acc_sh, sem):
    c = lax.axis_index("c")
    s = lax.axis_index("s")
    _zero_init(s, z_hbm, acc_sh)
    plsc.subcore_barrier()
    base = (c * _NS + s) * _EPT

    def chunk(k, carry):
        off = base + k * _C
        pltpu.sync_copy(src_hbm.at[pl.ds(off, _C)], src_v)
        pltpu.sync_copy(dst_hbm.at[pl.ds(off, _C)], dst_v)
        pltpu.async_copy(y_hbm.at[src_v], rows_v, sem).wait()
        pltpu.sync_copy(rows_v, acc_sh.at[dst_v], add=True)
        return carry

    lax.fori_loop(0, _NCHUNK, chunk, 0)
    plsc.subcore_barrier()
    _write_out(c, s, acc_sh, acc_out)


_sc_agg = pl.kernel(
    _sc_agg_body,
    out_type=jax.ShapeDtypeStruct((_NC * _N, _D), jnp.float32),
    mesh=_MESH,
    scratch_types=[
        pltpu.VMEM((_C,), jnp.int32),
        pltpu.VMEM((_C,), jnp.int32),
        pltpu.VMEM((_C, _D), jnp.float32),
        pltpu.VMEM_SHARED((_N, _D), jnp.float32),
        pltpu.SemaphoreType.DMA,
    ],
)


def _sc_deg_body(dst_hbm, z_hbm, ones_hbm, deg_out,
                 dst_v, ones_v, deg_sh):
    c = lax.axis_index("c")
    s = lax.axis_index("s")
    _zero_init(s, z_hbm, deg_sh)
    pltpu.sync_copy(ones_hbm, ones_v)
    plsc.subcore_barrier()
    base = (c * _NS + s) * _EPT

    def chunk(k, carry):
        off = base + k * _C
        pltpu.sync_copy(dst_hbm.at[pl.ds(off, _C)], dst_v)
        pltpu.sync_copy(ones_v, deg_sh.at[dst_v], add=True)
        return carry

    lax.fori_loop(0, _NCHUNK, chunk, 0)
    plsc.subcore_barrier()
    _write_out(c, s, deg_sh, deg_out)


_sc_deg = pl.kernel(
    _sc_deg_body,
    out_type=jax.ShapeDtypeStruct((_NC * _N, _D), jnp.float32),
    mesh=_MESH,
    scratch_types=[
        pltpu.VMEM((_C,), jnp.int32),
        pltpu.VMEM((_C, _D), jnp.float32),
        pltpu.VMEM_SHARED((_N, _D), jnp.float32),
    ],
)


def _dot_t(a, w):
    return lax.dot_general(a, w, (((1,), (1,)), ((), ())),
                           preferred_element_type=jnp.float32)


def _tc_pre_body(x_ref, wl_ref, wr_ref, y_ref, r_ref):
    xb = x_ref[...]
    y_ref[...] = _dot_t(xb, wl_ref[...])
    r_ref[...] = _dot_t(xb, wr_ref[...])


@jax.jit
def _tc_pre(x, wl, wr):
    return pl.pallas_call(
        _tc_pre_body,
        grid=(_G,),
        in_specs=[
            pl.BlockSpec((_BS, _D), lambda i: (i, 0)),
            pl.BlockSpec((_D, _D), lambda i: (0, 0)),
            pl.BlockSpec((_D, _D), lambda i: (0, 0)),
        ],
        out_specs=[
            pl.BlockSpec((_BS, _D), lambda i: (i, 0)),
            pl.BlockSpec((_BS, _D), lambda i: (i, 0)),
        ],
        out_shape=[
            jax.ShapeDtypeStruct((_N, _D), jnp.float32),
            jax.ShapeDtypeStruct((_N, _D), jnp.float32),
        ],
    )(x, wl, wr)


def _tc_mid_body(aA_ref, aB_ref, dA_ref, dB_ref, r1_ref, b_ref,
                 wl_ref, wr_ref, y2_ref, r2_ref):
    ssum = aA_ref[...] + aB_ref[...]
    deg = dA_ref[:, 0:1] + dB_ref[:, 0:1]
    inv = 1.0 / jnp.maximum(deg, 1.0)
    h = jnp.maximum(ssum * inv + b_ref[...] + r1_ref[...], 0.0)
    y2_ref[...] = _dot_t(h, wl_ref[...])
    r2_ref[...] = _dot_t(h, wr_ref[...])


@jax.jit
def _tc_mid(acc, deg, r1, b, wl, wr):
    return pl.pallas_call(
        _tc_mid_body,
        grid=(_G,),
        in_specs=[
            pl.BlockSpec((_BS, _D), lambda i: (i, 0)),
            pl.BlockSpec((_BS, _D), lambda i: (i + _G, 0)),
            pl.BlockSpec((_BS, _D), lambda i: (i, 0)),
            pl.BlockSpec((_BS, _D), lambda i: (i + _G, 0)),
            pl.BlockSpec((_BS, _D), lambda i: (i, 0)),
            pl.BlockSpec((1, _D), lambda i: (0, 0)),
            pl.BlockSpec((_D, _D), lambda i: (0, 0)),
            pl.BlockSpec((_D, _D), lambda i: (0, 0)),
        ],
        out_specs=[
            pl.BlockSpec((_BS, _D), lambda i: (i, 0)),
            pl.BlockSpec((_BS, _D), lambda i: (i, 0)),
        ],
        out_shape=[
            jax.ShapeDtypeStruct((_N, _D), jnp.float32),
            jax.ShapeDtypeStruct((_N, _D), jnp.float32),
        ],
    )(acc, acc, deg, deg, r1, b, wl, wr)


def _tc_final_body(aA_ref, aB_ref, dA_ref, dB_ref, r2_ref, b_ref, z_ref):
    ssum = aA_ref[...] + aB_ref[...]
    deg = dA_ref[:, 0:1] + dB_ref[:, 0:1]
    inv = 1.0 / jnp.maximum(deg, 1.0)
    z_ref[...] = ssum * inv + b_ref[...] + r2_ref[...]


@jax.jit
def _tc_final(acc, deg, r2, b):
    return pl.pallas_call(
        _tc_final_body,
        grid=(_G,),
        in_specs=[
            pl.BlockSpec((_BS, _D), lambda i: (i, 0)),
            pl.BlockSpec((_BS, _D), lambda i: (i + _G, 0)),
            pl.BlockSpec((_BS, _D), lambda i: (i, 0)),
            pl.BlockSpec((_BS, _D), lambda i: (i + _G, 0)),
            pl.BlockSpec((_BS, _D), lambda i: (i, 0)),
            pl.BlockSpec((1, _D), lambda i: (0, 0)),
        ],
        out_specs=pl.BlockSpec((_BS, _D), lambda i: (i, 0)),
        out_shape=jax.ShapeDtypeStruct((_N, _D), jnp.float32),
    )(acc, acc, deg, deg, r2, b)


def kernel(x, edge_index, W1l, b1l, W1r, W2l, b2l, W2r):
    src = edge_index[0].astype(jnp.int32)
    dst = edge_index[1].astype(jnp.int32)
    z128 = jnp.zeros((_RPT_LAST, _D), jnp.float32)
    ones128 = jnp.ones((_C, _D), jnp.float32)
    b1 = b1l.reshape(1, _D)
    b2 = b2l.reshape(1, _D)

    y1, r1 = _tc_pre(x, W1l, W1r)
    deg = _sc_deg(dst, z128, ones128)
    acc1 = _sc_agg(y1, src, dst, z128)
    y2, r2 = _tc_mid(acc1, deg, r1, b1, W2l, W2r)
    acc2 = _sc_agg(y2, src, dst, z128)
    z = _tc_final(acc2, deg, r2, b2)
    return z
